# baseline (speedup 1.0000x reference)
import functools

import jax
import jax.numpy as jnp
import numpy as np
from jax import lax
from jax.experimental import pallas as pl
from jax.experimental.pallas import tpu as pltpu
from jax.experimental.pallas import tpu_sc as plsc

_INTERPRET = False

N = 4096
D_CHL = 64
D_MODEL = 128
D_FFN = 256
N_HEADS = 4
DH = D_MODEL // N_HEADS
M = 16
DIST = 20.0
GRID = 128
PAD = 8
BR = 256
NBLK = N // BR
BIG = 1e9


def _proj_kernel(feat_ref, ci_ref, win_ref, bin_ref, wp1_ref, bp1_ref,
                 wp2_ref, bp2_ref, out_ref):
    src = jnp.dot(feat_ref[...], win_ref[...],
                  preferred_element_type=jnp.float32) + bin_ref[...]
    vc = ci_ref[...] * (1.0 / (GRID - 1))
    h = jnp.maximum(
        jnp.dot(vc, wp1_ref[...], preferred_element_type=jnp.float32)
        + bp1_ref[...], 0.0)
    pe = jnp.dot(h, wp2_ref[...],
                 preferred_element_type=jnp.float32) + bp2_ref[...]
    out_ref[0:N, :] = src + pe
    out_ref[N:N + PAD, :] = jnp.zeros((PAD, D_MODEL), jnp.float32)


def _proj(features, ci_pad, W_in, b_in, Wp1p, bp1, Wp2, bp2):
    return pl.pallas_call(
        _proj_kernel,
        out_shape=jax.ShapeDtypeStruct((N + PAD, D_MODEL), jnp.float32),
        interpret=_INTERPRET,
    )(features, ci_pad, W_in, b_in, Wp1p, bp1, Wp2, bp2)


def _topk_kernel(cr_ref, cc_ref, idx_ref):
    xr = cr_ref[:, 0:1]
    yr = cr_ref[:, 1:2]
    zr = cr_ref[:, 2:3]
    xc = cc_ref[0:1, :]
    yc = cc_ref[1:2, :]
    zc = cc_ref[2:3, :]
    manh = jnp.abs(xr - xc) + jnp.abs(yr - yc) + jnp.abs(zr - zc)
    col = lax.broadcasted_iota(jnp.int32, (BR, N), 1).astype(jnp.float32)
    key = jnp.where(manh <= DIST, manh * 4096.0 + col, BIG)
    cols = []
    for _ in range(M):
        m = jnp.min(key, axis=1, keepdims=True)
        key = jnp.where(key == m, BIG, key)
        mi = m.astype(jnp.int32)
        sel = jnp.where(m < BIG, jnp.bitwise_and(mi, 4095), N)
        cols.append(sel)
    idx_ref[...] = jnp.concatenate(cols, axis=1)


def _topk(ci_pad, ci_colsT):
    return pl.pallas_call(
        _topk_kernel,
        grid=(NBLK,),
        in_specs=[
            pl.BlockSpec((BR, 8), lambda i: (i, 0)),
            pl.BlockSpec((8, N), lambda i: (0, 0)),
        ],
        out_specs=pl.BlockSpec((BR, M), lambda i: (i, 0)),
        out_shape=jax.ShapeDtypeStruct((N, M), jnp.int32),
        compiler_params=pltpu.CompilerParams(
            dimension_semantics=("parallel",)),
        interpret=_INTERPRET,
    )(ci_pad, ci_colsT)


_NW = 32
_B_TOT = N * M
_B_PER_W = _B_TOT // _NW
_CHUNK = 64
_N_CHUNKS = _B_PER_W // _CHUNK


_NBUF = 4


def _gather(table, idx2d):
    mesh = plsc.VectorSubcoreMesh(core_axis_name="c", subcore_axis_name="s")

    @functools.partial(
        pl.kernel,
        mesh=mesh,
        out_type=jax.ShapeDtypeStruct((_B_TOT, D_MODEL), jnp.float32),
        scratch_types=[
            pltpu.VMEM((_N_CHUNKS, _CHUNK), jnp.int32),
            pltpu.VMEM((2 * _NBUF, _CHUNK, D_MODEL), jnp.float32),
            pltpu.SemaphoreType.DMA,
            pltpu.SemaphoreType.DMA,
        ],
    )
    def gk(table_hbm, idx_hbm, out_hbm, idx_v, rb, gsem, ssem):
        wid = lax.axis_index("s") * 2 + lax.axis_index("c")
        base = wid * _B_PER_W
        pltpu.sync_copy(idx_hbm.at[pl.ds(wid * _N_CHUNKS, _N_CHUNKS)], idx_v)
        pending = {0: [], 1: []}
        for g in range(_N_CHUNKS // _NBUF):
            s = g % 2
            for st in pending[s]:
                st.wait()
            gets = []
            for t in range(_NBUF):
                gets.append(pltpu.async_copy(
                    table_hbm.at[idx_v.at[g * _NBUF + t]],
                    rb.at[s * _NBUF + t], gsem))
            pending[s] = []
            for t in range(_NBUF):
                gets[t].wait()
                off = base + (g * _NBUF + t) * _CHUNK
                pending[s].append(pltpu.async_copy(
                    rb.at[s * _NBUF + t], out_hbm.at[pl.ds(off, _CHUNK)],
                    ssem))
        for s in (0, 1):
            for st in pending[s]:
                st.wait()

    return gk(table, idx2d)


def _ln(x, g, b):
    mu = jnp.mean(x, axis=-1, keepdims=True)
    var = jnp.mean((x - mu) * (x - mu), axis=-1, keepdims=True)
    return (x - mu) * jax.lax.rsqrt(var + 1e-5) * g + b


def _attn_kernel(nf_ref, src_ref, feat_ref, wq_ref, bq_ref, wk_ref, bk_ref,
                 wv_ref, bv_ref, wo_ref, bo_ref, g1_ref, b1_ref, wl1_ref,
                 bl1_ref, wl2_ref, bl2_ref, g3_ref, b3_ref, wf_ref, bf_ref,
                 out_ref):
    nfj = [nf_ref[j] for j in range(M)]
    nfall = jnp.concatenate(nfj, axis=0)
    q = jnp.dot(nfj[0], wq_ref[...],
                preferred_element_type=jnp.float32) + bq_ref[...]
    kk2 = (jnp.dot(nfall, wk_ref[...], preferred_element_type=jnp.float32)
           + bk_ref[...])
    vv2 = (jnp.dot(nfall, wv_ref[...], preferred_element_type=jnp.float32)
           + bv_ref[...])
    scale = 1.0 / np.sqrt(D_MODEL)
    qs = q * scale
    qtiles = [jnp.concatenate([qs[:, h * DH:(h + 1) * DH]] * 4, axis=1)
              for h in range(N_HEADS)]
    p = jnp.concatenate(
        [kk2[j * BR:(j + 1) * BR, :] * qtiles[j // 4] for j in range(M)],
        axis=0)
    ri = lax.broadcasted_iota(jnp.int32, (D_MODEL, D_MODEL), 0)
    cj = lax.broadcasted_iota(jnp.int32, (D_MODEL, D_MODEL), 1)
    eb = (jnp.right_shift(ri, 5) == jnp.right_shift(cj, 5)).astype(jnp.float32)
    lb = jnp.dot(p, eb, preferred_element_type=jnp.float32)
    lbj = [lb[j * BR:(j + 1) * BR, :] for j in range(M)]
    c4 = lambda x: jnp.concatenate([x] * 4, axis=1)
    sum4 = lambda x: (x[:, 0:DH] + x[:, DH:2 * DH]
                      + x[:, 2 * DH:3 * DH] + x[:, 3 * DH:4 * DH])
    hos = []
    for h in range(N_HEADS):
        l4 = lbj[4 * h:4 * h + 4]
        mp = jnp.maximum(jnp.maximum(l4[0], l4[1]),
                         jnp.maximum(l4[2], l4[3]))
        m32 = jnp.maximum(jnp.maximum(mp[:, 0:DH], mp[:, DH:2 * DH]),
                          jnp.maximum(mp[:, 2 * DH:3 * DH],
                                      mp[:, 3 * DH:4 * DH]))
        mfull = c4(m32)
        e4 = [jnp.exp(x - mfull) for x in l4]
        sp = (e4[0] + e4[1]) + (e4[2] + e4[3])
        rinv = c4(1.0 / sum4(sp))
        cs = None
        for a in range(4):
            t = (e4[a] * rinv) * vv2[(4 * h + a) * BR:(4 * h + a + 1) * BR, :]
            cs = t if cs is None else cs + t
        hos.append(sum4(cs))
    ho = jnp.concatenate(hos, axis=1)
    out = jnp.dot(ho, wo_ref[...],
                  preferred_element_type=jnp.float32) + bo_ref[...]
    tgt = _ln(src_ref[...] + out, g1_ref[...], b1_ref[...])
    t2 = jnp.dot(
        jnp.maximum(jnp.dot(tgt, wl1_ref[...],
                            preferred_element_type=jnp.float32)
                    + bl1_ref[...], 0.0),
        wl2_ref[...], preferred_element_type=jnp.float32) + bl2_ref[...]
    tgt = _ln(tgt + t2, g3_ref[...], b3_ref[...])
    fused = (jnp.dot(feat_ref[...], wf_ref[0:D_CHL, :],
                     preferred_element_type=jnp.float32)
             + jnp.dot(tgt, wf_ref[D_CHL:D_CHL + D_MODEL, :],
                       preferred_element_type=jnp.float32)
             + bf_ref[...])
    out_ref[...] = fused


def _attn(nf3, src_pad, features, Wq, bq, Wk, bk, Wv, bv, Wo, bo, g1, b1,
          Wl1, bl1, Wl2, bl2, g3, b3, Wf, bf):
    full = lambda r, c: pl.BlockSpec((r, c), lambda i: (0, 0))
    return pl.pallas_call(
        _attn_kernel,
        grid=(NBLK,),
        in_specs=[
            pl.BlockSpec((M, BR, D_MODEL), lambda i: (0, i, 0)),
            pl.BlockSpec((BR, D_MODEL), lambda i: (i, 0)),
            pl.BlockSpec((BR, D_CHL), lambda i: (i, 0)),
            full(D_MODEL, D_MODEL), full(1, D_MODEL),
            full(D_MODEL, D_MODEL), full(1, D_MODEL),
            full(D_MODEL, D_MODEL), full(1, D_MODEL),
            full(D_MODEL, D_MODEL), full(1, D_MODEL),
            full(1, D_MODEL), full(1, D_MODEL),
            full(D_MODEL, D_FFN), full(1, D_FFN),
            full(D_FFN, D_MODEL), full(1, D_MODEL),
            full(1, D_MODEL), full(1, D_MODEL),
            full(D_CHL + D_MODEL, D_CHL), full(1, D_CHL),
        ],
        out_specs=pl.BlockSpec((BR, D_CHL), lambda i: (i, 0)),
        out_shape=jax.ShapeDtypeStruct((N, D_CHL), jnp.float32),
        compiler_params=pltpu.CompilerParams(
            dimension_semantics=("parallel",)),
        interpret=_INTERPRET,
    )(nf3, src_pad, features, Wq, bq, Wk, bk, Wv, bv, Wo, bo, g1, b1,
      Wl1, bl1, Wl2, bl2, g3, b3, Wf, bf)


def _bn_kernel(f_ref, g_ref, b_ref, out_ref):
    f = f_ref[...]
    mu = jnp.mean(f, axis=0, keepdims=True)
    var = jnp.mean((f - mu) * (f - mu), axis=0, keepdims=True)
    y = (f - mu) * jax.lax.rsqrt(var + 1e-5) * g_ref[...] + b_ref[...]
    out_ref[...] = jnp.maximum(y, 0.0)


def _bn(fused, bn_g, bn_b):
    return pl.pallas_call(
        _bn_kernel,
        out_shape=jax.ShapeDtypeStruct((N, D_CHL), jnp.float32),
        interpret=_INTERPRET,
    )(fused, bn_g, bn_b)


def kernel(features, crt_indice, W_in, b_in, Wp1, bp1, Wp2, bp2, Wq, bq,
           Wk, bk, Wv, bv, Wo, bo, ln1_g, ln1_b, Wl1, bl1, Wl2, bl2,
           ln3_g, ln3_b, Wf, bf, bn_g, bn_b):
    ci_f = crt_indice.astype(jnp.float32)
    ci_pad = jnp.pad(ci_f, ((0, 0), (0, 5)))
    ci_colsT = ci_pad.T
    Wp1p = jnp.pad(Wp1, ((0, 5), (0, 0)))

    r1 = lambda x: x.reshape(1, -1)
    src_pad = _proj(features, ci_pad, W_in, r1(b_in), Wp1p, r1(bp1),
                    Wp2, r1(bp2))
    idx = _topk(ci_pad, ci_colsT)
    nf_flat = _gather(src_pad,
                      idx.T.reshape(-1, _CHUNK))
    nf3 = nf_flat.reshape(M, N, D_MODEL)
    fused = _attn(nf3, src_pad, features, Wq, r1(bq), Wk, r1(bk), Wv,
                  r1(bv), Wo, r1(bo), r1(ln1_g), r1(ln1_b), Wl1, r1(bl1),
                  Wl2, r1(bl2), r1(ln3_g), r1(ln3_b), Wf, r1(bf))
    return _bn(fused, r1(bn_g), r1(bn_b))


# speedup vs baseline: 1.4966x; 1.4966x over previous
import functools

import jax
import jax.numpy as jnp
import numpy as np
from jax import lax
from jax.experimental import pallas as pl
from jax.experimental.pallas import tpu as pltpu
from jax.experimental.pallas import tpu_sc as plsc

_INTERPRET = False

N = 4096
D_CHL = 64
D_MODEL = 128
D_FFN = 256
N_HEADS = 4
DH = D_MODEL // N_HEADS
M = 16
DIST = 20.0
GRID = 128
PAD = 128
BR = 256
NBLK = N // BR
BIG = 1e9


def _proj_kernel(feat_ref, ci_ref, win_ref, bin_ref, wp1_ref, bp1_ref,
                 wp2_ref, bp2_ref, out_ref):
    src = jnp.dot(feat_ref[...], win_ref[...],
                  preferred_element_type=jnp.float32) + bin_ref[...]
    vc = ci_ref[...] * (1.0 / (GRID - 1))
    h = jnp.maximum(
        jnp.dot(vc, wp1_ref[...], preferred_element_type=jnp.float32)
        + bp1_ref[...], 0.0)
    pe = jnp.dot(h, wp2_ref[...],
                 preferred_element_type=jnp.float32) + bp2_ref[...]
    out_ref[0:N, :] = src + pe
    out_ref[N:N + PAD, :] = jnp.zeros((PAD, D_MODEL), jnp.float32)


def _proj(features, ci_pad, W_in, b_in, Wp1p, bp1, Wp2, bp2):
    return pl.pallas_call(
        _proj_kernel,
        out_shape=jax.ShapeDtypeStruct((N + PAD, D_MODEL), jnp.float32),
        interpret=_INTERPRET,
    )(features, ci_pad, W_in, b_in, Wp1p, bp1, Wp2, bp2)


def _topk_kernel(cr_ref, cc_ref, idx_ref):
    xr = cr_ref[:, 0:1]
    yr = cr_ref[:, 1:2]
    zr = cr_ref[:, 2:3]
    xc = cc_ref[0:1, :]
    yc = cc_ref[1:2, :]
    zc = cc_ref[2:3, :]
    manh = jnp.abs(xr - xc) + jnp.abs(yr - yc) + jnp.abs(zr - zc)
    col = lax.broadcasted_iota(jnp.int32, (BR, N), 1).astype(jnp.float32)
    key = jnp.where(manh <= DIST, manh * 4096.0 + col, BIG)
    cols = []
    for _ in range(M):
        m = jnp.min(key, axis=1, keepdims=True)
        key = jnp.where(key == m, BIG, key)
        mi = m.astype(jnp.int32)
        sel = jnp.where(m < BIG, jnp.bitwise_and(mi, 4095), N)
        cols.append(sel)
    idx_ref[...] = jnp.concatenate(cols, axis=1)


def _topk(ci_pad, ci_colsT):
    return pl.pallas_call(
        _topk_kernel,
        grid=(NBLK,),
        in_specs=[
            pl.BlockSpec((BR, 8), lambda i: (i, 0)),
            pl.BlockSpec((8, N), lambda i: (0, 0)),
        ],
        out_specs=pl.BlockSpec((BR, M), lambda i: (i, 0)),
        out_shape=jax.ShapeDtypeStruct((N, M), jnp.int32),
        compiler_params=pltpu.CompilerParams(
            dimension_semantics=("parallel",)),
        interpret=_INTERPRET,
    )(ci_pad, ci_colsT)


_NW = 32
_B_TOT = N * M
_B_PER_W = _B_TOT // _NW
_CHUNK = 64
_N_CHUNKS = _B_PER_W // _CHUNK


_NBUF = 4


def _gather(table, idx2d):
    mesh = plsc.VectorSubcoreMesh(core_axis_name="c", subcore_axis_name="s")

    @functools.partial(
        pl.kernel,
        mesh=mesh,
        out_type=jax.ShapeDtypeStruct((_B_TOT, D_MODEL), jnp.float32),
        scratch_types=[
            pltpu.VMEM((_N_CHUNKS, _CHUNK), jnp.int32),
            pltpu.VMEM((2 * _NBUF, _CHUNK, D_MODEL), jnp.float32),
            pltpu.VMEM_SHARED((N + PAD, D_MODEL), jnp.float32),
            pltpu.SemaphoreType.DMA,
            pltpu.SemaphoreType.DMA,
        ],
    )
    def gk(table_hbm, idx_hbm, out_hbm, idx_v, rb, shared, gsem, ssem):
        cid = lax.axis_index("c")
        sid = lax.axis_index("s")
        wid = sid * 2 + cid
        base = wid * _B_PER_W
        rows_per_sub = (N + PAD) // 16
        pltpu.sync_copy(table_hbm.at[pl.ds(sid * rows_per_sub, rows_per_sub)],
                        shared.at[pl.ds(sid * rows_per_sub, rows_per_sub)])
        pltpu.sync_copy(idx_hbm.at[pl.ds(wid * _N_CHUNKS, _N_CHUNKS)], idx_v)
        plsc.subcore_barrier()
        pending = {0: [], 1: []}
        for g in range(_N_CHUNKS // _NBUF):
            s = g % 2
            for st in pending[s]:
                st.wait()
            gets = []
            for t in range(_NBUF):
                gets.append(pltpu.async_copy(
                    shared.at[idx_v.at[g * _NBUF + t]],
                    rb.at[s * _NBUF + t], gsem))
            pending[s] = []
            for t in range(_NBUF):
                gets[t].wait()
                off = base + (g * _NBUF + t) * _CHUNK
                pending[s].append(pltpu.async_copy(
                    rb.at[s * _NBUF + t], out_hbm.at[pl.ds(off, _CHUNK)],
                    ssem))
        for s in (0, 1):
            for st in pending[s]:
                st.wait()

    return gk(table, idx2d)


def _ln(x, g, b):
    mu = jnp.mean(x, axis=-1, keepdims=True)
    var = jnp.mean((x - mu) * (x - mu), axis=-1, keepdims=True)
    return (x - mu) * jax.lax.rsqrt(var + 1e-5) * g + b


def _attn_kernel(nf_ref, src_ref, feat_ref, wq_ref, bq_ref, wk_ref, bk_ref,
                 wv_ref, bv_ref, wo_ref, bo_ref, g1_ref, b1_ref, wl1_ref,
                 bl1_ref, wl2_ref, bl2_ref, g3_ref, b3_ref, wf_ref, bf_ref,
                 out_ref):
    nfj = [nf_ref[j] for j in range(M)]
    nfall = jnp.concatenate(nfj, axis=0)
    q = jnp.dot(nfj[0], wq_ref[...],
                preferred_element_type=jnp.float32) + bq_ref[...]
    kk2 = (jnp.dot(nfall, wk_ref[...], preferred_element_type=jnp.float32)
           + bk_ref[...])
    vv2 = (jnp.dot(nfall, wv_ref[...], preferred_element_type=jnp.float32)
           + bv_ref[...])
    scale = 1.0 / np.sqrt(D_MODEL)
    qs = q * scale
    qtiles = [jnp.concatenate([qs[:, h * DH:(h + 1) * DH]] * 4, axis=1)
              for h in range(N_HEADS)]
    p = jnp.concatenate(
        [kk2[j * BR:(j + 1) * BR, :] * qtiles[j // 4] for j in range(M)],
        axis=0)
    ri = lax.broadcasted_iota(jnp.int32, (D_MODEL, D_MODEL), 0)
    cj = lax.broadcasted_iota(jnp.int32, (D_MODEL, D_MODEL), 1)
    eb = (jnp.right_shift(ri, 5) == jnp.right_shift(cj, 5)).astype(jnp.float32)
    lb = jnp.dot(p, eb, preferred_element_type=jnp.float32)
    lbj = [lb[j * BR:(j + 1) * BR, :] for j in range(M)]
    c4 = lambda x: jnp.concatenate([x] * 4, axis=1)
    sum4 = lambda x: (x[:, 0:DH] + x[:, DH:2 * DH]
                      + x[:, 2 * DH:3 * DH] + x[:, 3 * DH:4 * DH])
    hos = []
    for h in range(N_HEADS):
        l4 = lbj[4 * h:4 * h + 4]
        mp = jnp.maximum(jnp.maximum(l4[0], l4[1]),
                         jnp.maximum(l4[2], l4[3]))
        m32 = jnp.maximum(jnp.maximum(mp[:, 0:DH], mp[:, DH:2 * DH]),
                          jnp.maximum(mp[:, 2 * DH:3 * DH],
                                      mp[:, 3 * DH:4 * DH]))
        mfull = c4(m32)
        e4 = [jnp.exp(x - mfull) for x in l4]
        sp = (e4[0] + e4[1]) + (e4[2] + e4[3])
        rinv = c4(1.0 / sum4(sp))
        cs = None
        for a in range(4):
            t = (e4[a] * rinv) * vv2[(4 * h + a) * BR:(4 * h + a + 1) * BR, :]
            cs = t if cs is None else cs + t
        hos.append(sum4(cs))
    ho = jnp.concatenate(hos, axis=1)
    out = jnp.dot(ho, wo_ref[...],
                  preferred_element_type=jnp.float32) + bo_ref[...]
    tgt = _ln(src_ref[...] + out, g1_ref[...], b1_ref[...])
    t2 = jnp.dot(
        jnp.maximum(jnp.dot(tgt, wl1_ref[...],
                            preferred_element_type=jnp.float32)
                    + bl1_ref[...], 0.0),
        wl2_ref[...], preferred_element_type=jnp.float32) + bl2_ref[...]
    tgt = _ln(tgt + t2, g3_ref[...], b3_ref[...])
    fused = (jnp.dot(feat_ref[...], wf_ref[0:D_CHL, :],
                     preferred_element_type=jnp.float32)
             + jnp.dot(tgt, wf_ref[D_CHL:D_CHL + D_MODEL, :],
                       preferred_element_type=jnp.float32)
             + bf_ref[...])
    out_ref[...] = fused


def _attn(nf3, src_pad, features, Wq, bq, Wk, bk, Wv, bv, Wo, bo, g1, b1,
          Wl1, bl1, Wl2, bl2, g3, b3, Wf, bf):
    full = lambda r, c: pl.BlockSpec((r, c), lambda i: (0, 0))
    return pl.pallas_call(
        _attn_kernel,
        grid=(NBLK,),
        in_specs=[
            pl.BlockSpec((M, BR, D_MODEL), lambda i: (0, i, 0)),
            pl.BlockSpec((BR, D_MODEL), lambda i: (i, 0)),
            pl.BlockSpec((BR, D_CHL), lambda i: (i, 0)),
            full(D_MODEL, D_MODEL), full(1, D_MODEL),
            full(D_MODEL, D_MODEL), full(1, D_MODEL),
            full(D_MODEL, D_MODEL), full(1, D_MODEL),
            full(D_MODEL, D_MODEL), full(1, D_MODEL),
            full(1, D_MODEL), full(1, D_MODEL),
            full(D_MODEL, D_FFN), full(1, D_FFN),
            full(D_FFN, D_MODEL), full(1, D_MODEL),
            full(1, D_MODEL), full(1, D_MODEL),
            full(D_CHL + D_MODEL, D_CHL), full(1, D_CHL),
        ],
        out_specs=pl.BlockSpec((BR, D_CHL), lambda i: (i, 0)),
        out_shape=jax.ShapeDtypeStruct((N, D_CHL), jnp.float32),
        compiler_params=pltpu.CompilerParams(
            dimension_semantics=("parallel",)),
        interpret=_INTERPRET,
    )(nf3, src_pad, features, Wq, bq, Wk, bk, Wv, bv, Wo, bo, g1, b1,
      Wl1, bl1, Wl2, bl2, g3, b3, Wf, bf)


def _bn_kernel(f_ref, g_ref, b_ref, out_ref):
    f = f_ref[...]
    mu = jnp.mean(f, axis=0, keepdims=True)
    var = jnp.mean((f - mu) * (f - mu), axis=0, keepdims=True)
    y = (f - mu) * jax.lax.rsqrt(var + 1e-5) * g_ref[...] + b_ref[...]
    out_ref[...] = jnp.maximum(y, 0.0)


def _bn(fused, bn_g, bn_b):
    return pl.pallas_call(
        _bn_kernel,
        out_shape=jax.ShapeDtypeStruct((N, D_CHL), jnp.float32),
        interpret=_INTERPRET,
    )(fused, bn_g, bn_b)


def kernel(features, crt_indice, W_in, b_in, Wp1, bp1, Wp2, bp2, Wq, bq,
           Wk, bk, Wv, bv, Wo, bo, ln1_g, ln1_b, Wl1, bl1, Wl2, bl2,
           ln3_g, ln3_b, Wf, bf, bn_g, bn_b):
    ci_f = crt_indice.astype(jnp.float32)
    ci_pad = jnp.pad(ci_f, ((0, 0), (0, 5)))
    ci_colsT = ci_pad.T
    Wp1p = jnp.pad(Wp1, ((0, 5), (0, 0)))

    r1 = lambda x: x.reshape(1, -1)
    src_pad = _proj(features, ci_pad, W_in, r1(b_in), Wp1p, r1(bp1),
                    Wp2, r1(bp2))
    idx = _topk(ci_pad, ci_colsT)
    nf_flat = _gather(src_pad,
                      idx.T.reshape(-1, _CHUNK))
    nf3 = nf_flat.reshape(M, N, D_MODEL)
    fused = _attn(nf3, src_pad, features, Wq, r1(bq), Wk, r1(bk), Wv,
                  r1(bv), Wo, r1(bo), r1(ln1_g), r1(ln1_b), Wl1, r1(bl1),
                  Wl2, r1(bl2), r1(ln3_g), r1(ln3_b), Wf, r1(bf))
    return _bn(fused, r1(bn_g), r1(bn_b))
